# Initial kernel scaffold; baseline (speedup 1.0000x reference)
import jax
import jax.numpy as jnp
from jax.experimental import pallas as pl


def kernel(input_ids, prompt_pos, word_table, prompt_table, W1, b1, W2, b2, pos_table, type_table, ln_gamma, ln_beta):
    raise NotImplementedError("write your pallas kernel here")


# speedup vs baseline: 1.7650x; 1.7650x over previous
import functools

import jax
import jax.numpy as jnp
from jax import lax
from jax.experimental import pallas as pl
from jax.experimental.pallas import tpu as pltpu
from jax.experimental.pallas import tpu_sc as plsc

_B, _S, _H, _V, _P = 64, 512, 1024, 50265, 50
_PAD = 1
_EPS = 1e-5

_NC, _NS = 2, 16
_NW = _NC * _NS
_ROWS = _B * _S
_RPW = _ROWS // _NW
_CH = 64
_NCHUNK = _RPW // _CH

_sc_mesh = plsc.VectorSubcoreMesh(core_axis_name="c", subcore_axis_name="s")


@functools.partial(
    pl.kernel,
    mesh=_sc_mesh,
    out_type=jax.ShapeDtypeStruct((_ROWS, _H), jnp.float32),
    scratch_types=[
        pltpu.VMEM((_CH,), jnp.int32),
        pltpu.VMEM((_CH, _H), jnp.float32),
        pltpu.SemaphoreType.DMA,
    ],
)
def _sc_gather(ids_hbm, table_hbm, out_hbm, idx_v, rows_v, sem):
    wid = lax.axis_index("s") * _NC + lax.axis_index("c")
    base = wid * _RPW

    def body(c, carry):
        o = base + c * _CH
        pltpu.sync_copy(ids_hbm.at[pl.ds(o, _CH)], idx_v)
        pltpu.async_copy(table_hbm.at[idx_v], rows_v, sem).wait()
        pltpu.sync_copy(rows_v, out_hbm.at[pl.ds(o, _CH)])
        return carry

    lax.fori_loop(0, _NCHUNK, body, 0)


def _mlp_body(p_ref, w1_ref, b1_ref, w2_ref, b2_ref, o_ref):
    h = jnp.dot(p_ref[...], w1_ref[...], preferred_element_type=jnp.float32)
    h = jnp.maximum(h + b1_ref[...], 0.0)
    o_ref[...] = jnp.dot(h, w2_ref[...], preferred_element_type=jnp.float32) + b2_ref[...]


_BLK = 256


def _combine_body(raw_ref, pos_ref, pe_ref, type_ref, g_ref, b_ref, o_ref):
    i = pl.program_id(0)
    r = lax.broadcasted_iota(jnp.int32, (_BLK, 1), 0)
    s_lo = (i % (_S // _BLK)) * _BLK
    mask = (r + s_lo) < _P
    x = jnp.where(mask, pe_ref[...], raw_ref[...])
    x = x + pos_ref[...] + type_ref[...]
    mean = jnp.mean(x, axis=1, keepdims=True)
    cent = x - mean
    var = jnp.mean(cent * cent, axis=1, keepdims=True)
    o_ref[...] = cent * lax.rsqrt(var + _EPS) * g_ref[...] + b_ref[...]


def kernel(input_ids, prompt_pos, word_table, prompt_table, W1, b1, W2, b2,
           pos_table, type_table, ln_gamma, ln_beta):
    ids_flat = input_ids.reshape(_ROWS).astype(jnp.int32)

    raw = _sc_gather(ids_flat, word_table)

    p_pad = jnp.zeros((_BLK, _H), jnp.float32).at[:_P].set(prompt_table)
    pe = pl.pallas_call(
        _mlp_body,
        out_shape=jax.ShapeDtypeStruct((_BLK, _H), jnp.float32),
    )(p_pad, W1, b1.reshape(1, _H), W2, b2.reshape(1, _H))

    pos_slice = lax.slice(pos_table, (_PAD + 1, 0), (_PAD + 1 + _S, _H))
    nblk = _ROWS // _BLK
    sblk = _S // _BLK
    out = pl.pallas_call(
        _combine_body,
        grid=(nblk,),
        in_specs=[
            pl.BlockSpec((_BLK, _H), lambda i: (i, 0)),
            pl.BlockSpec((_BLK, _H), lambda i: (i % sblk, 0)),
            pl.BlockSpec((_BLK, _H), lambda i: (0, 0)),
            pl.BlockSpec((1, _H), lambda i: (0, 0)),
            pl.BlockSpec((1, _H), lambda i: (0, 0)),
            pl.BlockSpec((1, _H), lambda i: (0, 0)),
        ],
        out_specs=pl.BlockSpec((_BLK, _H), lambda i: (i, 0)),
        out_shape=jax.ShapeDtypeStruct((_ROWS, _H), jnp.float32),
    )(raw, pos_slice, pe, type_table, ln_gamma.reshape(1, _H),
      ln_beta.reshape(1, _H))

    return out.reshape(_B, _S, _H)


# speedup vs baseline: 2.2216x; 1.2586x over previous
import functools

import jax
import jax.numpy as jnp
from jax import lax
from jax.experimental import pallas as pl
from jax.experimental.pallas import tpu as pltpu
from jax.experimental.pallas import tpu_sc as plsc

_B, _S, _H, _V, _P = 64, 512, 1024, 50265, 50
_PAD = 1
_EPS = 1e-5

_NC, _NS = 2, 16
_NW = _NC * _NS
_ROWS = _B * _S
_RPW = _ROWS // _NW
_CH = 64
_NCHUNK = _RPW // _CH

_sc_mesh = plsc.VectorSubcoreMesh(core_axis_name="c", subcore_axis_name="s")


@functools.partial(
    pl.kernel,
    mesh=_sc_mesh,
    out_type=jax.ShapeDtypeStruct((_ROWS, _H), jnp.float32),
    scratch_types=[
        pltpu.VMEM((_CH,), jnp.int32),
        pltpu.VMEM((_CH, _H), jnp.float32),
        pltpu.SemaphoreType.DMA,
    ],
)
def _sc_gather(ids_hbm, table_hbm, out_hbm, idx_v, rows_v, sem):
    wid = lax.axis_index("s") * _NC + lax.axis_index("c")
    base = wid * _RPW

    def body(c, carry):
        o = base + c * _CH
        pltpu.sync_copy(ids_hbm.at[pl.ds(o, _CH)], idx_v)
        pltpu.async_copy(table_hbm.at[idx_v], rows_v, sem).wait()
        pltpu.sync_copy(rows_v, out_hbm.at[pl.ds(o, _CH)])
        return carry

    lax.fori_loop(0, _NCHUNK, body, 0)


def _mlp_body(p_ref, w1_ref, b1_ref, w2_ref, b2_ref, o_ref):
    h = jnp.dot(p_ref[...], w1_ref[...], preferred_element_type=jnp.float32)
    h = jnp.maximum(h + b1_ref[...], 0.0)
    o_ref[...] = jnp.dot(h, w2_ref[...], preferred_element_type=jnp.float32) + b2_ref[...]


_BLK = 512


def _combine_body(raw_ref, pos_ref, pe_ref, type_ref, g_ref, b_ref, o_ref):
    r = lax.broadcasted_iota(jnp.int32, (_BLK, 1), 0)
    mask = r < _P
    x = jnp.where(mask, pe_ref[...], raw_ref[...])
    x = x + pos_ref[...] + type_ref[...]
    mean = jnp.mean(x, axis=1, keepdims=True)
    cent = x - mean
    var = jnp.mean(cent * cent, axis=1, keepdims=True)
    o_ref[...] = cent * lax.rsqrt(var + _EPS) * g_ref[...] + b_ref[...]


def kernel(input_ids, prompt_pos, word_table, prompt_table, W1, b1, W2, b2,
           pos_table, type_table, ln_gamma, ln_beta):
    ids_flat = input_ids.reshape(_ROWS).astype(jnp.int32)

    raw = _sc_gather(ids_flat, word_table)

    p_pad = jnp.zeros((_BLK, _H), jnp.float32).at[:_P].set(prompt_table)
    pe = pl.pallas_call(
        _mlp_body,
        out_shape=jax.ShapeDtypeStruct((_BLK, _H), jnp.float32),
    )(p_pad, W1, b1.reshape(1, _H), W2, b2.reshape(1, _H))

    pos_slice = lax.slice(pos_table, (_PAD + 1, 0), (_PAD + 1 + _S, _H))
    nblk = _ROWS // _BLK
    out = pl.pallas_call(
        _combine_body,
        grid=(nblk,),
        in_specs=[
            pl.BlockSpec((_BLK, _H), lambda i: (i, 0)),
            pl.BlockSpec((_BLK, _H), lambda i: (0, 0)),
            pl.BlockSpec((_BLK, _H), lambda i: (0, 0)),
            pl.BlockSpec((1, _H), lambda i: (0, 0)),
            pl.BlockSpec((1, _H), lambda i: (0, 0)),
            pl.BlockSpec((1, _H), lambda i: (0, 0)),
        ],
        out_specs=pl.BlockSpec((_BLK, _H), lambda i: (i, 0)),
        out_shape=jax.ShapeDtypeStruct((_ROWS, _H), jnp.float32),
    )(raw, pos_slice, pe, type_table, ln_gamma.reshape(1, _H),
      ln_beta.reshape(1, _H))

    return out.reshape(_B, _S, _H)


# speedup vs baseline: 2.3472x; 1.0566x over previous
import functools

import jax
import jax.numpy as jnp
from jax import lax
from jax.experimental import pallas as pl
from jax.experimental.pallas import tpu as pltpu
from jax.experimental.pallas import tpu_sc as plsc

_B, _S, _H, _V, _P = 64, 512, 1024, 50265, 50
_PAD = 1
_EPS = 1e-5

_NC, _NS = 2, 16
_NW = _NC * _NS
_ROWS = _B * _S
_RPW = _ROWS // _NW
_CH = 32
_NCHUNK = _RPW // _CH

_sc_mesh = plsc.VectorSubcoreMesh(core_axis_name="c", subcore_axis_name="s")


@functools.partial(
    pl.kernel,
    mesh=_sc_mesh,
    out_type=jax.ShapeDtypeStruct((_ROWS, _H), jnp.float32),
    scratch_types=[
        pltpu.VMEM((_RPW,), jnp.int32),
        pltpu.VMEM((_CH, _H), jnp.float32),
        pltpu.VMEM((_CH, _H), jnp.float32),
        pltpu.SemaphoreType.DMA,
        pltpu.SemaphoreType.DMA,
    ],
)
def _sc_gather(ids_hbm, table_hbm, out_hbm, idx_v, rows0_v, rows1_v, sem0, sem1):
    wid = lax.axis_index("s") * _NC + lax.axis_index("c")
    base = wid * _RPW

    def gather(c, buf, sem):
        return pltpu.make_async_copy(
            table_hbm.at[idx_v.at[pl.ds(c * _CH, _CH)]], buf, sem)

    def writeback(c, buf):
        pltpu.sync_copy(buf, out_hbm.at[pl.ds(base + c * _CH, _CH)])

    pltpu.sync_copy(ids_hbm.at[pl.ds(base, _RPW)], idx_v)
    gather(0, rows0_v, sem0).start()

    def body(k, carry):
        c0 = 2 * k
        gather(c0 + 1, rows1_v, sem1).start()
        gather(c0, rows0_v, sem0).wait()
        writeback(c0, rows0_v)

        @pl.when(k < _NCHUNK // 2 - 1)
        def _():
            gather(c0 + 2, rows0_v, sem0).start()

        gather(c0 + 1, rows1_v, sem1).wait()
        writeback(c0 + 1, rows1_v)
        return carry

    lax.fori_loop(0, _NCHUNK // 2, body, 0)


def _mlp_body(p_ref, w1_ref, b1_ref, w2_ref, b2_ref, o_ref):
    h = jnp.dot(p_ref[...], w1_ref[...], preferred_element_type=jnp.float32)
    h = jnp.maximum(h + b1_ref[...], 0.0)
    o_ref[...] = jnp.dot(h, w2_ref[...], preferred_element_type=jnp.float32) + b2_ref[...]


_BLK = 512


def _combine_body(raw_ref, pos_ref, pe_ref, type_ref, g_ref, b_ref, o_ref):
    r = lax.broadcasted_iota(jnp.int32, (_BLK, 1), 0)
    mask = r < _P
    x = jnp.where(mask, pe_ref[...], raw_ref[...])
    x = x + pos_ref[...] + type_ref[...]
    mean = jnp.mean(x, axis=1, keepdims=True)
    cent = x - mean
    var = jnp.mean(cent * cent, axis=1, keepdims=True)
    o_ref[...] = cent * lax.rsqrt(var + _EPS) * g_ref[...] + b_ref[...]


def kernel(input_ids, prompt_pos, word_table, prompt_table, W1, b1, W2, b2,
           pos_table, type_table, ln_gamma, ln_beta):
    ids_flat = input_ids.reshape(_ROWS).astype(jnp.int32)

    raw = _sc_gather(ids_flat, word_table)

    p_pad = jnp.zeros((_BLK, _H), jnp.float32).at[:_P].set(prompt_table)
    pe = pl.pallas_call(
        _mlp_body,
        out_shape=jax.ShapeDtypeStruct((_BLK, _H), jnp.float32),
    )(p_pad, W1, b1.reshape(1, _H), W2, b2.reshape(1, _H))

    pos_slice = lax.slice(pos_table, (_PAD + 1, 0), (_PAD + 1 + _S, _H))
    nblk = _ROWS // _BLK
    out = pl.pallas_call(
        _combine_body,
        grid=(nblk,),
        in_specs=[
            pl.BlockSpec((_BLK, _H), lambda i: (i, 0)),
            pl.BlockSpec((_BLK, _H), lambda i: (0, 0)),
            pl.BlockSpec((_BLK, _H), lambda i: (0, 0)),
            pl.BlockSpec((1, _H), lambda i: (0, 0)),
            pl.BlockSpec((1, _H), lambda i: (0, 0)),
            pl.BlockSpec((1, _H), lambda i: (0, 0)),
        ],
        out_specs=pl.BlockSpec((_BLK, _H), lambda i: (i, 0)),
        out_shape=jax.ShapeDtypeStruct((_ROWS, _H), jnp.float32),
    )(raw, pos_slice, pe, type_table, ln_gamma.reshape(1, _H),
      ln_beta.reshape(1, _H))

    return out.reshape(_B, _S, _H)


# speedup vs baseline: 2.4130x; 1.0280x over previous
import functools

import jax
import jax.numpy as jnp
from jax import lax
from jax.experimental import pallas as pl
from jax.experimental.pallas import tpu as pltpu
from jax.experimental.pallas import tpu_sc as plsc

_B, _S, _H, _V, _P = 64, 512, 1024, 50265, 50
_PAD = 1
_EPS = 1e-5

_NC, _NS = 2, 16
_NW = _NC * _NS
_ROWS = _B * _S
_NSPLIT = 4
_ROWS_C = _ROWS // _NSPLIT
_RPW = _ROWS_C // _NW
_CH = 32
_NCHUNK = _RPW // _CH

_sc_mesh = plsc.VectorSubcoreMesh(core_axis_name="c", subcore_axis_name="s")


@functools.partial(
    pl.kernel,
    mesh=_sc_mesh,
    out_type=jax.ShapeDtypeStruct((_ROWS_C, _H), jnp.float32),
    scratch_types=[
        pltpu.VMEM((_RPW,), jnp.int32),
        pltpu.VMEM((_CH, _H), jnp.float32),
        pltpu.VMEM((_CH, _H), jnp.float32),
        pltpu.SemaphoreType.DMA,
        pltpu.SemaphoreType.DMA,
    ],
)
def _sc_gather(ids_hbm, table_hbm, out_hbm, idx_v, rows0_v, rows1_v, sem0, sem1):
    wid = lax.axis_index("s") * _NC + lax.axis_index("c")
    base = wid * _RPW

    def gather(c, buf, sem):
        return pltpu.make_async_copy(
            table_hbm.at[idx_v.at[pl.ds(c * _CH, _CH)]], buf, sem)

    def writeback(c, buf):
        pltpu.sync_copy(buf, out_hbm.at[pl.ds(base + c * _CH, _CH)])

    pltpu.sync_copy(ids_hbm.at[pl.ds(base, _RPW)], idx_v)
    gather(0, rows0_v, sem0).start()

    def body(k, carry):
        c0 = 2 * k
        gather(c0 + 1, rows1_v, sem1).start()
        gather(c0, rows0_v, sem0).wait()
        writeback(c0, rows0_v)

        @pl.when(k < _NCHUNK // 2 - 1)
        def _():
            gather(c0 + 2, rows0_v, sem0).start()

        gather(c0 + 1, rows1_v, sem1).wait()
        writeback(c0 + 1, rows1_v)
        return carry

    lax.fori_loop(0, _NCHUNK // 2, body, 0)


def _mlp_body(p_ref, w1_ref, b1_ref, w2_ref, b2_ref, o_ref):
    h = jnp.dot(p_ref[...], w1_ref[...], preferred_element_type=jnp.float32)
    h = jnp.maximum(h + b1_ref[...], 0.0)
    o_ref[...] = jnp.dot(h, w2_ref[...], preferred_element_type=jnp.float32) + b2_ref[...]


_BLK = 512


def _ln_combine(raw_ref, pos_ref, pe_ref, type_ref, g_ref, b_ref, o_ref):
    r = lax.broadcasted_iota(jnp.int32, (_BLK, 1), 0)
    mask = r < _P
    x = jnp.where(mask, pe_ref[...], raw_ref[...])
    x = x + pos_ref[...] + type_ref[...]
    mean = jnp.mean(x, axis=1, keepdims=True)
    cent = x - mean
    var = jnp.mean(cent * cent, axis=1, keepdims=True)
    o_ref[...] = cent * lax.rsqrt(var + _EPS) * g_ref[...] + b_ref[...]


def _combine_first(raw_ref, pos_ref, pe_ref, type_ref, g_ref, b_ref, o_ref):
    _ln_combine(raw_ref, pos_ref, pe_ref, type_ref, g_ref, b_ref, o_ref)


def _combine_chained(raw_ref, pos_ref, pe_ref, type_ref, g_ref, b_ref,
                     prev_ref, o_ref):
    _ln_combine(raw_ref, pos_ref, pe_ref, type_ref, g_ref, b_ref, o_ref)


_BPC = _ROWS_C // _BLK


def _combine_call(c, raw_c, pos_slice, pe, type_table, g2d, b2d, prev):
    base_specs = [
        pl.BlockSpec((_BLK, _H), lambda i: (i, 0)),
        pl.BlockSpec((_BLK, _H), lambda i: (0, 0)),
        pl.BlockSpec((_BLK, _H), lambda i: (0, 0)),
        pl.BlockSpec((1, _H), lambda i: (0, 0)),
        pl.BlockSpec((1, _H), lambda i: (0, 0)),
        pl.BlockSpec((1, _H), lambda i: (0, 0)),
    ]
    out_spec = pl.BlockSpec((_BLK, _H), lambda i, c=c: (c * _BPC + i, 0))
    out_shape = jax.ShapeDtypeStruct((_ROWS, _H), jnp.float32)
    args = (raw_c, pos_slice, pe, type_table, g2d, b2d)
    if prev is None:
        return pl.pallas_call(
            _combine_first, grid=(_BPC,), in_specs=base_specs,
            out_specs=out_spec, out_shape=out_shape)(*args)
    return pl.pallas_call(
        _combine_chained, grid=(_BPC,),
        in_specs=base_specs + [pl.BlockSpec(memory_space=pl.ANY)],
        out_specs=out_spec, out_shape=out_shape,
        input_output_aliases={6: 0})(*args, prev)


def kernel(input_ids, prompt_pos, word_table, prompt_table, W1, b1, W2, b2,
           pos_table, type_table, ln_gamma, ln_beta):
    ids_flat = input_ids.reshape(_ROWS).astype(jnp.int32)

    p_pad = jnp.zeros((_BLK, _H), jnp.float32).at[:_P].set(prompt_table)
    pe = pl.pallas_call(
        _mlp_body,
        out_shape=jax.ShapeDtypeStruct((_BLK, _H), jnp.float32),
    )(p_pad, W1, b1.reshape(1, _H), W2, b2.reshape(1, _H))

    pos_slice = lax.slice(pos_table, (_PAD + 1, 0), (_PAD + 1 + _S, _H))
    g2d = ln_gamma.reshape(1, _H)
    b2d = ln_beta.reshape(1, _H)

    out = None
    for c in range(_NSPLIT):
        ids_c = lax.slice(ids_flat, (c * _ROWS_C,), ((c + 1) * _ROWS_C,))
        raw_c = _sc_gather(ids_c, word_table)
        out = _combine_call(c, raw_c, pos_slice, pe, type_table, g2d, b2d, out)

    return out.reshape(_B, _S, _H)
